# baseline (speedup 1.0000x reference)
import functools

import jax
import jax.numpy as jnp
from jax import lax
from jax.experimental import pallas as pl
from jax.experimental.pallas import tpu as pltpu
from jax.experimental.pallas import tpu_sc as plsc

S = 20480
SEQ_LEN = 10
H_DIM = 128
EMBD_RANK = 64
N = 320000
B = 2048
N_ENT = 10000
N_REL = 500

NC = 2
NS = 16

AC = 80
SROWS = 20608
SLAB = SROWS // NS
MBLK = 56
NMB = SLAB // MBLK
RPT = N // NS
RB1 = 40
NRB = RPT // RB1
NG1 = NRB // 3
RB2 = 80
QPT = (B * SEQ_LEN) // NS
NQB = QPT // RB2


def _body(emb, sids, dt_ext, ent_ext, rel_ext, idxf, eidx, ridx, onespad,
          zer, full, dtout, acc, rb0, rb1, rb2, ib0, ib1, ib2, rbufP, idxP,
          mbuf, dtb, ebuf, eixb, sL0, sL1, sL2, sA0, sA1, sA2, semG):
    cid = lax.axis_index("c")
    sid = lax.axis_index("s")
    col0 = cid * 64
    rbs = (rb0, rb1, rb2)
    ibs = (ib0, ib1, ib2)
    sLs = (sL0, sL1, sL2)
    sAs = (sA0, sA1, sA2)

    pltpu.sync_copy(zer, mbuf)
    for b in range(NMB):
        pltpu.sync_copy(mbuf, acc.at[pl.ds(sid * SLAB + b * MBLK, MBLK), :])
    for rb in rbs:
        pltpu.sync_copy(onespad, rb.at[:, 64:AC])
    plsc.subcore_barrier()

    def loads(b, i):
        r0 = sid * RPT + i * RB1
        return (
            pltpu.make_async_copy(
                emb.at[pl.ds(r0, RB1), pl.ds(col0, 64)], rbs[b].at[:, 0:64],
                sLs[b]),
            pltpu.make_async_copy(sids.at[pl.ds(r0, RB1)], ibs[b], sLs[b]),
        )

    def start_loads(b, i):
        for d in loads(b, i):
            d.start()

    def wait_loads(b, i):
        for d in loads(b, i):
            d.wait()

    def start_add(b):
        pltpu.async_copy(rbs[b], acc.at[ibs[b]], sAs[b], add=True)

    def wait_add(b):
        pltpu.make_async_copy(rbs[b], acc.at[ibs[b]], sAs[b]).wait()

    start_loads(0, 0)
    start_loads(1, 1)

    def grp(g, _):
        for t in range(3):
            i = 3 * g + t
            wait_loads(t, i)
            start_add(t)
            if t == 0:
                @pl.when(g > 0)
                def _():
                    wait_add(2)
            else:
                wait_add(t - 1)
            start_loads((t + 2) % 3, i + 2)
        return 0

    lax.fori_loop(0, NG1, grp, 0)
    wait_loads(0, NRB - 2)
    start_add(0)
    wait_add(2)
    wait_loads(1, NRB - 1)
    start_add(1)
    wait_add(0)
    wait_add(1)
    plsc.subcore_barrier()

    def scale_row(r, _):
        v = mbuf[r, 64:80]
        rv = 1.0 / jnp.maximum(v, 1.0)
        s = rv[0]
        for j in range(4):
            mbuf[r, j * 16:(j + 1) * 16] = mbuf[r, j * 16:(j + 1) * 16] * s
        mbuf[r, 64:80] = dtb[pl.ds(r, 16)]
        return 0

    for b in range(NMB):
        row0 = sid * SLAB + b * MBLK
        pltpu.sync_copy(acc.at[pl.ds(row0, MBLK), :], mbuf)
        pltpu.sync_copy(dt_ext.at[pl.ds(row0, MBLK)], dtb.at[pl.ds(0, MBLK)])
        lax.fori_loop(0, MBLK, scale_row, 0)
        pltpu.sync_copy(mbuf, acc.at[pl.ds(row0, MBLK), :])
    plsc.subcore_barrier()

    def p2(k, _):
        k0 = sid * QPT + k * RB2
        pltpu.sync_copy(idxf.at[pl.ds(k0, RB2)], idxP)
        pltpu.async_copy(acc.at[idxP], rbufP, semG).wait()

        @pl.when(cid == 0)
        def _():
            pltpu.sync_copy(rbufP.at[:, 0:64], full.at[pl.ds(k0, RB2), 0:64])
            pltpu.sync_copy(rbufP.at[:, 64:65], dtout.at[pl.ds(k0, RB2), :])
            pltpu.sync_copy(eidx.at[pl.ds(k0, RB2)], eixb)
            pltpu.async_copy(ent_ext.at[eixb], ebuf, semG).wait()
            pltpu.sync_copy(ebuf, full.at[pl.ds(k0, RB2), 128:192])

        @pl.when(cid == 1)
        def _():
            pltpu.sync_copy(rbufP.at[:, 0:64], full.at[pl.ds(k0, RB2), 64:128])
            pltpu.sync_copy(ridx.at[pl.ds(k0, RB2)], eixb)
            pltpu.async_copy(rel_ext.at[eixb], ebuf, semG).wait()
            pltpu.sync_copy(ebuf, full.at[pl.ds(k0, RB2), 192:256])

        return 0

    lax.fori_loop(0, NQB, p2, 0)


_sc_call = functools.partial(
    pl.kernel,
    out_type=(
        jax.ShapeDtypeStruct((B * SEQ_LEN, 256), jnp.float32),
        jax.ShapeDtypeStruct((B * SEQ_LEN, 1), jnp.float32),
    ),
    mesh=plsc.VectorSubcoreMesh(core_axis_name="c", subcore_axis_name="s"),
    compiler_params=pltpu.CompilerParams(use_tc_tiling_on_sc=False),
    scratch_types=[
        pltpu.VMEM_SHARED((SROWS, AC), jnp.float32),
        pltpu.VMEM((RB1, AC), jnp.float32),
        pltpu.VMEM((RB1, AC), jnp.float32),
        pltpu.VMEM((RB1, AC), jnp.float32),
        pltpu.VMEM((RB1,), jnp.int32),
        pltpu.VMEM((RB1,), jnp.int32),
        pltpu.VMEM((RB1,), jnp.int32),
        pltpu.VMEM((RB2, AC), jnp.float32),
        pltpu.VMEM((RB2,), jnp.int32),
        pltpu.VMEM((MBLK, AC), jnp.float32),
        pltpu.VMEM((MBLK + 16,), jnp.float32),
        pltpu.VMEM((RB2, EMBD_RANK), jnp.float32),
        pltpu.VMEM((RB2,), jnp.int32),
        pltpu.SemaphoreType.DMA,
        pltpu.SemaphoreType.DMA,
        pltpu.SemaphoreType.DMA,
        pltpu.SemaphoreType.DMA,
        pltpu.SemaphoreType.DMA,
        pltpu.SemaphoreType.DMA,
        pltpu.SemaphoreType.DMA,
    ],
)(_body)


def kernel(embeds_stack, ent_embeds, rel_embeds, dt_vals, segment_ids,
           s_len_non_zero, s_tem, r_tem):
    lens = s_len_non_zero.astype(jnp.int32)
    offsets = jnp.concatenate(
        [jnp.zeros((1,), jnp.int32), jnp.cumsum(lens)[:-1]])
    pos = jnp.arange(SEQ_LEN, dtype=jnp.int32)
    idx = offsets[:, None] + pos[None, :]
    mask = pos[None, :] < lens[:, None]
    idxf = jnp.where(mask, idx, S).reshape(-1).astype(jnp.int32)
    eidx = jnp.where(mask, s_tem.astype(jnp.int32)[:, None],
                     N_ENT).reshape(-1)
    ridx = jnp.where(mask, r_tem.astype(jnp.int32)[:, None],
                     N_REL).reshape(-1)

    dt_ext = jnp.zeros((SROWS,), jnp.float32).at[:S].set(dt_vals)
    ent_ext = jnp.zeros((N_ENT + 8, EMBD_RANK), jnp.float32).at[:N_ENT].set(
        ent_embeds)
    rel_ext = jnp.zeros((N_REL + 8, EMBD_RANK), jnp.float32).at[:N_REL].set(
        rel_embeds)
    onespad = jnp.zeros((RB1, AC - 64), jnp.float32).at[:, 0].set(1.0)
    zer = jnp.zeros((MBLK, AC), jnp.float32)

    full, dt = _sc_call(embeds_stack, segment_ids.astype(jnp.int32), dt_ext,
                        ent_ext, rel_ext, idxf, eidx, ridx, onespad, zer)
    return full.reshape(B, SEQ_LEN, 256), dt.reshape(B, SEQ_LEN)


# speedup vs baseline: 1.0274x; 1.0274x over previous
import functools

import jax
import jax.numpy as jnp
from jax import lax
from jax.experimental import pallas as pl
from jax.experimental.pallas import tpu as pltpu
from jax.experimental.pallas import tpu_sc as plsc

S = 20480
SEQ_LEN = 10
H_DIM = 128
EMBD_RANK = 64
N = 320000
B = 2048
N_ENT = 10000
N_REL = 500

NC = 2
NS = 16

AC = 80
SROWS = 20608
SLAB = SROWS // NS
MBLK = 56
NMB = SLAB // MBLK
RPT = N // NS
RB1 = 80
NRB = RPT // RB1
NPAIR = NRB // 2
RB2 = 80
QPT = (B * SEQ_LEN) // NS
NQB = QPT // RB2


def _body(emb, sids, dt_ext, ent_ext, rel_ext, idxf, eidx, ridx, onespad,
          zer, full, dtout, acc, rbufA, rbufB, idxbA, idxbB, mbuf, dtb,
          ebuf, eixb, semL, semAA, semAB, semG2):
    cid = lax.axis_index("c")
    sid = lax.axis_index("s")
    col0 = cid * 64

    pltpu.sync_copy(zer, mbuf)
    for b in range(NMB):
        pltpu.sync_copy(mbuf, acc.at[pl.ds(sid * SLAB + b * MBLK, MBLK), :])
    pltpu.sync_copy(onespad, rbufA.at[:, 64:AC])
    pltpu.sync_copy(onespad, rbufB.at[:, 64:AC])
    plsc.subcore_barrier()

    def loads(buf, ib, i):
        r0 = sid * RPT + i * RB1
        return (
            pltpu.make_async_copy(
                emb.at[pl.ds(r0, RB1), pl.ds(col0, 64)], buf.at[:, 0:64],
                semL),
            pltpu.make_async_copy(sids.at[pl.ds(r0, RB1)], ib, semL),
        )

    def start_loads(buf, ib, i):
        for d in loads(buf, ib, i):
            d.start()

    def wait_loads(buf, ib, i):
        for d in loads(buf, ib, i):
            d.wait()

    start_loads(rbufA, idxbA, 0)

    def pair(j, _):
        @pl.when(j > 0)
        def _():
            pltpu.make_async_copy(rbufB, acc.at[idxbB], semAB).wait()

        start_loads(rbufB, idxbB, 2 * j + 1)
        wait_loads(rbufA, idxbA, 2 * j)
        pltpu.async_copy(rbufA, acc.at[idxbA], semAA, add=True)
        pltpu.make_async_copy(rbufA, acc.at[idxbA], semAA).wait()

        @pl.when(j < NPAIR - 1)
        def _():
            start_loads(rbufA, idxbA, 2 * j + 2)

        wait_loads(rbufB, idxbB, 2 * j + 1)
        pltpu.async_copy(rbufB, acc.at[idxbB], semAB, add=True)
        return 0

    lax.fori_loop(0, NPAIR, pair, 0)
    pltpu.make_async_copy(rbufB, acc.at[idxbB], semAB).wait()
    plsc.subcore_barrier()

    def scale_row(r, _):
        v = mbuf[r, 64:80]
        rv = 1.0 / jnp.maximum(v, 1.0)
        s = rv[0]
        for j in range(4):
            mbuf[r, j * 16:(j + 1) * 16] = mbuf[r, j * 16:(j + 1) * 16] * s
        mbuf[r, 64:80] = dtb[pl.ds(r, 16)]
        return 0

    for b in range(NMB):
        row0 = sid * SLAB + b * MBLK
        pltpu.sync_copy(acc.at[pl.ds(row0, MBLK), :], mbuf)
        pltpu.sync_copy(dt_ext.at[pl.ds(row0, MBLK)], dtb.at[pl.ds(0, MBLK)])
        lax.fori_loop(0, MBLK, scale_row, 0)
        pltpu.sync_copy(mbuf, acc.at[pl.ds(row0, MBLK), :])
    plsc.subcore_barrier()

    def p2(k, _):
        k0 = sid * QPT + k * RB2
        pltpu.sync_copy(idxf.at[pl.ds(k0, RB2)], idxbA)
        gm = pltpu.async_copy(acc.at[idxbA], rbufA, semL)

        @pl.when(cid == 0)
        def _():
            pltpu.sync_copy(eidx.at[pl.ds(k0, RB2)], eixb)
            pltpu.async_copy(ent_ext.at[eixb], ebuf, semG2)

        @pl.when(cid == 1)
        def _():
            pltpu.sync_copy(ridx.at[pl.ds(k0, RB2)], eixb)
            pltpu.async_copy(rel_ext.at[eixb], ebuf, semG2)

        gm.wait()

        @pl.when(cid == 0)
        def _():
            pltpu.sync_copy(rbufA.at[:, 0:64], full.at[pl.ds(k0, RB2), 0:64])
            pltpu.sync_copy(rbufA.at[:, 64:65], dtout.at[pl.ds(k0, RB2), :])
            pltpu.make_async_copy(ent_ext.at[eixb], ebuf, semG2).wait()
            pltpu.sync_copy(ebuf, full.at[pl.ds(k0, RB2), 128:192])

        @pl.when(cid == 1)
        def _():
            pltpu.sync_copy(rbufA.at[:, 0:64], full.at[pl.ds(k0, RB2), 64:128])
            pltpu.make_async_copy(rel_ext.at[eixb], ebuf, semG2).wait()
            pltpu.sync_copy(ebuf, full.at[pl.ds(k0, RB2), 192:256])

        return 0

    lax.fori_loop(0, NQB, p2, 0)


_sc_call = functools.partial(
    pl.kernel,
    out_type=(
        jax.ShapeDtypeStruct((B * SEQ_LEN, 256), jnp.float32),
        jax.ShapeDtypeStruct((B * SEQ_LEN, 1), jnp.float32),
    ),
    mesh=plsc.VectorSubcoreMesh(core_axis_name="c", subcore_axis_name="s"),
    compiler_params=pltpu.CompilerParams(use_tc_tiling_on_sc=False),
    scratch_types=[
        pltpu.VMEM_SHARED((SROWS, AC), jnp.float32),
        pltpu.VMEM((RB1, AC), jnp.float32),
        pltpu.VMEM((RB1, AC), jnp.float32),
        pltpu.VMEM((RB1,), jnp.int32),
        pltpu.VMEM((RB1,), jnp.int32),
        pltpu.VMEM((MBLK, AC), jnp.float32),
        pltpu.VMEM((MBLK + 16,), jnp.float32),
        pltpu.VMEM((RB2, EMBD_RANK), jnp.float32),
        pltpu.VMEM((RB2,), jnp.int32),
        pltpu.SemaphoreType.DMA,
        pltpu.SemaphoreType.DMA,
        pltpu.SemaphoreType.DMA,
        pltpu.SemaphoreType.DMA,
    ],
)(_body)


def kernel(embeds_stack, ent_embeds, rel_embeds, dt_vals, segment_ids,
           s_len_non_zero, s_tem, r_tem):
    lens = s_len_non_zero.astype(jnp.int32)
    offsets = jnp.concatenate(
        [jnp.zeros((1,), jnp.int32), jnp.cumsum(lens)[:-1]])
    pos = jnp.arange(SEQ_LEN, dtype=jnp.int32)
    idx = offsets[:, None] + pos[None, :]
    mask = pos[None, :] < lens[:, None]
    idxf = jnp.where(mask, idx, S).reshape(-1).astype(jnp.int32)
    eidx = jnp.where(mask, s_tem.astype(jnp.int32)[:, None],
                     N_ENT).reshape(-1)
    ridx = jnp.where(mask, r_tem.astype(jnp.int32)[:, None],
                     N_REL).reshape(-1)

    dt_ext = jnp.zeros((SROWS,), jnp.float32).at[:S].set(dt_vals)
    ent_ext = jnp.zeros((N_ENT + 8, EMBD_RANK), jnp.float32).at[:N_ENT].set(
        ent_embeds)
    rel_ext = jnp.zeros((N_REL + 8, EMBD_RANK), jnp.float32).at[:N_REL].set(
        rel_embeds)
    onespad = jnp.zeros((RB1, AC - 64), jnp.float32).at[:, 0].set(1.0)
    zer = jnp.zeros((MBLK, AC), jnp.float32)

    full, dt = _sc_call(embeds_stack, segment_ids.astype(jnp.int32), dt_ext,
                        ent_ext, rel_ext, idxf, eidx, ridx, onespad, zer)
    return full.reshape(B, SEQ_LEN, 256), dt.reshape(B, SEQ_LEN)


# speedup vs baseline: 1.0335x; 1.0059x over previous
import functools

import jax
import jax.numpy as jnp
from jax import lax
from jax.experimental import pallas as pl
from jax.experimental.pallas import tpu as pltpu
from jax.experimental.pallas import tpu_sc as plsc

S = 20480
SEQ_LEN = 10
H_DIM = 128
EMBD_RANK = 64
N = 320000
B = 2048
N_ENT = 10000
N_REL = 500

NC = 2
NS = 16

AC = 72
SROWS = 20608
SLAB = SROWS // NS
MBLK = 56
NMB = SLAB // MBLK
RPT = N // NS
RB1 = 80
NRB = RPT // RB1
NPAIR = NRB // 2
RB2 = 80
QPT = (B * SEQ_LEN) // NS
NQB = QPT // RB2


def _body(emb, sids, dt_ext, ent_ext, rel_ext, idxf, eidx, ridx, onespad,
          zer, full, dtout, acc, rbufA, rbufB, idxbA, idxbB, mbuf, dtb,
          ebuf, eixb, semL, semAA, semAB, semG2):
    cid = lax.axis_index("c")
    sid = lax.axis_index("s")
    col0 = cid * 64

    pltpu.sync_copy(zer, mbuf)
    for b in range(NMB):
        pltpu.sync_copy(mbuf, acc.at[pl.ds(sid * SLAB + b * MBLK, MBLK), :])
    pltpu.sync_copy(onespad, rbufA.at[:, 64:AC])
    pltpu.sync_copy(onespad, rbufB.at[:, 64:AC])
    plsc.subcore_barrier()

    def loads(buf, ib, i):
        r0 = sid * RPT + i * RB1
        return (
            pltpu.make_async_copy(
                emb.at[pl.ds(r0, RB1), pl.ds(col0, 64)], buf.at[:, 0:64],
                semL),
            pltpu.make_async_copy(sids.at[pl.ds(r0, RB1)], ib, semL),
        )

    def start_loads(buf, ib, i):
        for d in loads(buf, ib, i):
            d.start()

    def wait_loads(buf, ib, i):
        for d in loads(buf, ib, i):
            d.wait()

    start_loads(rbufA, idxbA, 0)

    def pair(j, _):
        @pl.when(j > 0)
        def _():
            pltpu.make_async_copy(rbufB, acc.at[idxbB], semAB).wait()

        start_loads(rbufB, idxbB, 2 * j + 1)
        wait_loads(rbufA, idxbA, 2 * j)
        pltpu.async_copy(rbufA, acc.at[idxbA], semAA, add=True)
        pltpu.make_async_copy(rbufA, acc.at[idxbA], semAA).wait()

        @pl.when(j < NPAIR - 1)
        def _():
            start_loads(rbufA, idxbA, 2 * j + 2)

        wait_loads(rbufB, idxbB, 2 * j + 1)
        pltpu.async_copy(rbufB, acc.at[idxbB], semAB, add=True)
        return 0

    lax.fori_loop(0, NPAIR, pair, 0)
    pltpu.make_async_copy(rbufB, acc.at[idxbB], semAB).wait()
    plsc.subcore_barrier()

    lanes = lax.iota(jnp.int32, 16)

    def scale_row(r, _):
        v = mbuf[r, 56:72]
        rv = 1.0 / jnp.maximum(v, 1.0)
        s = rv[8]
        for j in range(4):
            mbuf[r, j * 16:(j + 1) * 16] = mbuf[r, j * 16:(j + 1) * 16] * s
        w = mbuf[r, 56:72]
        dtv = dtb[pl.ds(r, 16)]
        mbuf[r, 56:72] = jnp.where(lanes < 8, w, dtv)
        return 0

    for b in range(NMB):
        row0 = sid * SLAB + b * MBLK
        pltpu.sync_copy(acc.at[pl.ds(row0, MBLK), :], mbuf)
        pltpu.sync_copy(dt_ext.at[pl.ds(row0, MBLK)], dtb.at[pl.ds(8, MBLK)])
        lax.fori_loop(0, MBLK, scale_row, 0)
        pltpu.sync_copy(mbuf, acc.at[pl.ds(row0, MBLK), :])
    plsc.subcore_barrier()

    def p2(k, _):
        k0 = sid * QPT + k * RB2
        pltpu.sync_copy(idxf.at[pl.ds(k0, RB2)], idxbA)
        gm = pltpu.async_copy(acc.at[idxbA], rbufA, semL)

        @pl.when(cid == 0)
        def _():
            pltpu.sync_copy(eidx.at[pl.ds(k0, RB2)], eixb)
            pltpu.async_copy(ent_ext.at[eixb], ebuf, semG2)

        @pl.when(cid == 1)
        def _():
            pltpu.sync_copy(ridx.at[pl.ds(k0, RB2)], eixb)
            pltpu.async_copy(rel_ext.at[eixb], ebuf, semG2)

        gm.wait()

        @pl.when(cid == 0)
        def _():
            pltpu.sync_copy(rbufA.at[:, 0:64], full.at[pl.ds(k0, RB2), 0:64])
            pltpu.sync_copy(rbufA.at[:, 64:65], dtout.at[pl.ds(k0, RB2), :])
            pltpu.make_async_copy(ent_ext.at[eixb], ebuf, semG2).wait()
            pltpu.sync_copy(ebuf, full.at[pl.ds(k0, RB2), 128:192])

        @pl.when(cid == 1)
        def _():
            pltpu.sync_copy(rbufA.at[:, 0:64], full.at[pl.ds(k0, RB2), 64:128])
            pltpu.make_async_copy(rel_ext.at[eixb], ebuf, semG2).wait()
            pltpu.sync_copy(ebuf, full.at[pl.ds(k0, RB2), 192:256])

        return 0

    lax.fori_loop(0, NQB, p2, 0)


_sc_call = functools.partial(
    pl.kernel,
    out_type=(
        jax.ShapeDtypeStruct((B * SEQ_LEN, 256), jnp.float32),
        jax.ShapeDtypeStruct((B * SEQ_LEN, 1), jnp.float32),
    ),
    mesh=plsc.VectorSubcoreMesh(core_axis_name="c", subcore_axis_name="s"),
    compiler_params=pltpu.CompilerParams(use_tc_tiling_on_sc=False),
    scratch_types=[
        pltpu.VMEM_SHARED((SROWS, AC), jnp.float32),
        pltpu.VMEM((RB1, AC), jnp.float32),
        pltpu.VMEM((RB1, AC), jnp.float32),
        pltpu.VMEM((RB1,), jnp.int32),
        pltpu.VMEM((RB1,), jnp.int32),
        pltpu.VMEM((MBLK, AC), jnp.float32),
        pltpu.VMEM((MBLK + 24,), jnp.float32),
        pltpu.VMEM((RB2, EMBD_RANK), jnp.float32),
        pltpu.VMEM((RB2,), jnp.int32),
        pltpu.SemaphoreType.DMA,
        pltpu.SemaphoreType.DMA,
        pltpu.SemaphoreType.DMA,
        pltpu.SemaphoreType.DMA,
    ],
)(_body)


def kernel(embeds_stack, ent_embeds, rel_embeds, dt_vals, segment_ids,
           s_len_non_zero, s_tem, r_tem):
    lens = s_len_non_zero.astype(jnp.int32)
    offsets = jnp.concatenate(
        [jnp.zeros((1,), jnp.int32), jnp.cumsum(lens)[:-1]])
    pos = jnp.arange(SEQ_LEN, dtype=jnp.int32)
    idx = offsets[:, None] + pos[None, :]
    mask = pos[None, :] < lens[:, None]
    idxf = jnp.where(mask, idx, S).reshape(-1).astype(jnp.int32)
    eidx = jnp.where(mask, s_tem.astype(jnp.int32)[:, None],
                     N_ENT).reshape(-1)
    ridx = jnp.where(mask, r_tem.astype(jnp.int32)[:, None],
                     N_REL).reshape(-1)

    dt_ext = jnp.zeros((SROWS,), jnp.float32).at[:S].set(dt_vals)
    ent_ext = jnp.zeros((N_ENT + 8, EMBD_RANK), jnp.float32).at[:N_ENT].set(
        ent_embeds)
    rel_ext = jnp.zeros((N_REL + 8, EMBD_RANK), jnp.float32).at[:N_REL].set(
        rel_embeds)
    onespad = jnp.zeros((RB1, AC - 64), jnp.float32).at[:, 0].set(1.0)
    zer = jnp.zeros((MBLK, AC), jnp.float32)

    full, dt = _sc_call(embeds_stack, segment_ids.astype(jnp.int32), dt_ext,
                        ent_ext, rel_ext, idxf, eidx, ridx, onespad, zer)
    return full.reshape(B, SEQ_LEN, 256), dt.reshape(B, SEQ_LEN)


# speedup vs baseline: 1.0451x; 1.0112x over previous
import functools

import jax
import jax.numpy as jnp
from jax import lax
from jax.experimental import pallas as pl
from jax.experimental.pallas import tpu as pltpu
from jax.experimental.pallas import tpu_sc as plsc

S = 20480
SEQ_LEN = 10
H_DIM = 128
EMBD_RANK = 64
N = 320000
B = 2048
N_ENT = 10000
N_REL = 500

NC = 2
NS = 16

AC = 72
SROWS = 20608
SLAB = SROWS // NS
MBLK = 56
NMB = SLAB // MBLK
RPT = N // NS
RB1 = 80
NRB = RPT // RB1
NPAIR = NRB // 2
RB2 = 80
QPT = (B * SEQ_LEN) // NS
NQB = QPT // RB2


def _body(emb, sids, dt_ext, ent_ext, rel_ext, idxf, eidx, ridx, onespad,
          zer, full, dtout, acc, rbufA, rbufB, idxbA, idxbB, mbuf, dtb,
          ebuf, eixb, semL, semAA, semAB, semG2):
    cid = lax.axis_index("c")
    sid = lax.axis_index("s")
    col0 = cid * 64

    pltpu.sync_copy(zer, acc.at[pl.ds(sid * SLAB, SLAB), :])
    pltpu.sync_copy(onespad, rbufA.at[:, 64:AC])
    pltpu.sync_copy(onespad, rbufB.at[:, 64:AC])
    plsc.subcore_barrier()

    def loads(buf, ib, i):
        r0 = sid * RPT + i * RB1
        return (
            pltpu.make_async_copy(
                emb.at[pl.ds(r0, RB1), pl.ds(col0, 64)], buf.at[:, 0:64],
                semL),
            pltpu.make_async_copy(sids.at[pl.ds(r0, RB1)], ib, semL),
        )

    def start_loads(buf, ib, i):
        for d in loads(buf, ib, i):
            d.start()

    def wait_loads(buf, ib, i):
        for d in loads(buf, ib, i):
            d.wait()

    start_loads(rbufA, idxbA, 0)

    def pair(j, _):
        @pl.when(j > 0)
        def _():
            pltpu.make_async_copy(rbufB, acc.at[idxbB], semAB).wait()

        start_loads(rbufB, idxbB, 2 * j + 1)
        wait_loads(rbufA, idxbA, 2 * j)
        pltpu.async_copy(rbufA, acc.at[idxbA], semAA, add=True)
        pltpu.make_async_copy(rbufA, acc.at[idxbA], semAA).wait()

        @pl.when(j < NPAIR - 1)
        def _():
            start_loads(rbufA, idxbA, 2 * j + 2)

        wait_loads(rbufB, idxbB, 2 * j + 1)
        pltpu.async_copy(rbufB, acc.at[idxbB], semAB, add=True)
        return 0

    lax.fori_loop(0, NPAIR, pair, 0)
    pltpu.make_async_copy(rbufB, acc.at[idxbB], semAB).wait()
    plsc.subcore_barrier()

    lanes = lax.iota(jnp.int32, 16)

    def scale_row(r, _):
        v = mbuf[r, 56:72]
        rv = 1.0 / jnp.maximum(v, 1.0)
        s = rv[8]
        for j in range(4):
            mbuf[r, j * 16:(j + 1) * 16] = mbuf[r, j * 16:(j + 1) * 16] * s
        w = mbuf[r, 56:72]
        dtv = dtb[pl.ds(r, 16)]
        mbuf[r, 56:72] = jnp.where(lanes < 8, w, dtv)
        return 0

    for b in range(NMB):
        row0 = sid * SLAB + b * MBLK
        dm = pltpu.async_copy(acc.at[pl.ds(row0, MBLK), :], mbuf, semL)
        dd = pltpu.async_copy(dt_ext.at[pl.ds(row0, MBLK)],
                              dtb.at[pl.ds(8, MBLK)], semG2)
        dm.wait()
        dd.wait()
        lax.fori_loop(0, MBLK, scale_row, 0)
        pltpu.sync_copy(mbuf, acc.at[pl.ds(row0, MBLK), :])
    plsc.subcore_barrier()

    def p2(k, _):
        k0 = sid * QPT + k * RB2
        pltpu.sync_copy(idxf.at[pl.ds(k0, RB2)], idxbA)
        gm = pltpu.async_copy(acc.at[idxbA], rbufA, semL)

        @pl.when(cid == 0)
        def _():
            pltpu.sync_copy(eidx.at[pl.ds(k0, RB2)], eixb)
            pltpu.async_copy(ent_ext.at[eixb], ebuf, semG2)

        @pl.when(cid == 1)
        def _():
            pltpu.sync_copy(ridx.at[pl.ds(k0, RB2)], eixb)
            pltpu.async_copy(rel_ext.at[eixb], ebuf, semG2)

        gm.wait()

        @pl.when(cid == 0)
        def _():
            pltpu.sync_copy(rbufA.at[:, 0:64], full.at[pl.ds(k0, RB2), 0:64])
            pltpu.sync_copy(rbufA.at[:, 64:65], dtout.at[pl.ds(k0, RB2), :])
            pltpu.make_async_copy(ent_ext.at[eixb], ebuf, semG2).wait()
            pltpu.sync_copy(ebuf, full.at[pl.ds(k0, RB2), 128:192])

        @pl.when(cid == 1)
        def _():
            pltpu.sync_copy(rbufA.at[:, 0:64], full.at[pl.ds(k0, RB2), 64:128])
            pltpu.make_async_copy(rel_ext.at[eixb], ebuf, semG2).wait()
            pltpu.sync_copy(ebuf, full.at[pl.ds(k0, RB2), 192:256])

        return 0

    lax.fori_loop(0, NQB, p2, 0)


_sc_call = functools.partial(
    pl.kernel,
    out_type=(
        jax.ShapeDtypeStruct((B * SEQ_LEN, 256), jnp.float32),
        jax.ShapeDtypeStruct((B * SEQ_LEN, 1), jnp.float32),
    ),
    mesh=plsc.VectorSubcoreMesh(core_axis_name="c", subcore_axis_name="s"),
    compiler_params=pltpu.CompilerParams(use_tc_tiling_on_sc=False),
    scratch_types=[
        pltpu.VMEM_SHARED((SROWS, AC), jnp.float32),
        pltpu.VMEM((RB1, AC), jnp.float32),
        pltpu.VMEM((RB1, AC), jnp.float32),
        pltpu.VMEM((RB1,), jnp.int32),
        pltpu.VMEM((RB1,), jnp.int32),
        pltpu.VMEM((MBLK, AC), jnp.float32),
        pltpu.VMEM((MBLK + 24,), jnp.float32),
        pltpu.VMEM((RB2, EMBD_RANK), jnp.float32),
        pltpu.VMEM((RB2,), jnp.int32),
        pltpu.SemaphoreType.DMA,
        pltpu.SemaphoreType.DMA,
        pltpu.SemaphoreType.DMA,
        pltpu.SemaphoreType.DMA,
    ],
)(_body)


def kernel(embeds_stack, ent_embeds, rel_embeds, dt_vals, segment_ids,
           s_len_non_zero, s_tem, r_tem):
    lens = s_len_non_zero.astype(jnp.int32)
    offsets = jnp.concatenate(
        [jnp.zeros((1,), jnp.int32), jnp.cumsum(lens)[:-1]])
    pos = jnp.arange(SEQ_LEN, dtype=jnp.int32)
    idx = offsets[:, None] + pos[None, :]
    mask = pos[None, :] < lens[:, None]
    idxf = jnp.where(mask, idx, S).reshape(-1).astype(jnp.int32)
    eidx = jnp.where(mask, s_tem.astype(jnp.int32)[:, None],
                     N_ENT).reshape(-1)
    ridx = jnp.where(mask, r_tem.astype(jnp.int32)[:, None],
                     N_REL).reshape(-1)

    dt_ext = jnp.zeros((SROWS,), jnp.float32).at[:S].set(dt_vals)
    ent_ext = jnp.zeros((N_ENT + 8, EMBD_RANK), jnp.float32).at[:N_ENT].set(
        ent_embeds)
    rel_ext = jnp.zeros((N_REL + 8, EMBD_RANK), jnp.float32).at[:N_REL].set(
        rel_embeds)
    onespad = jnp.zeros((RB1, AC - 64), jnp.float32).at[:, 0].set(1.0)
    zer = jnp.zeros((SLAB, AC), jnp.float32)

    full, dt = _sc_call(embeds_stack, segment_ids.astype(jnp.int32), dt_ext,
                        ent_ext, rel_ext, idxf, eidx, ridx, onespad, zer)
    return full.reshape(B, SEQ_LEN, 256), dt.reshape(B, SEQ_LEN)
